# baseline (speedup 1.0000x reference)
import functools

import jax
import jax.numpy as jnp
from jax import lax
from jax.experimental import pallas as pl
from jax.experimental.pallas import tpu as pltpu
from jax.experimental.pallas import tpu_sc as plsc

_LANES = 16


@jax.jit
def _sc_gather(t_idx, table):
    batch = t_idx.shape[0]
    table_size = table.shape[0]
    info = plsc.get_sparse_core_info()
    num_workers = info.num_subcores
    per_worker = batch // num_workers
    half = per_worker // 2

    mesh = plsc.VectorSubcoreMesh(
        core_axis_name="c", subcore_axis_name="s", num_cores=1
    )

    @functools.partial(
        pl.kernel,
        mesh=mesh,
        out_type=jax.ShapeDtypeStruct((batch,), jnp.float32),
        compiler_params=pltpu.CompilerParams(needs_layout_passes=False),
        scratch_types=[
            pltpu.VMEM((per_worker,), jnp.int32),
            pltpu.VMEM((table_size,), jnp.float32),
            pltpu.VMEM((per_worker,), jnp.float32),
            pltpu.SemaphoreType.DMA,
            pltpu.SemaphoreType.DMA,
        ],
    )
    def gather_kernel(
        t_hbm, table_hbm, out_hbm, idx_v, table_v, out_v, sem_in, sem_out
    ):
        wid = lax.axis_index("s")
        base = wid * per_worker
        cp_idx = pltpu.make_async_copy(
            t_hbm.at[pl.ds(base, per_worker)], idx_v, sem_in
        )
        cp_tab = pltpu.make_async_copy(table_hbm, table_v, sem_in)
        cp_idx.start()
        cp_tab.start()
        cp_idx.wait()
        cp_tab.wait()

        out_copies = []
        for h in range(2):
            @plsc.parallel_loop(h * half, (h + 1) * half, _LANES, unroll=8)
            def body(off):
                idx_vec = idx_v[pl.ds(off, _LANES)]
                out_v[pl.ds(off, _LANES)] = plsc.load_gather(
                    table_v, [idx_vec]
                )

            cp = pltpu.make_async_copy(
                out_v.at[pl.ds(h * half, half)],
                out_hbm.at[pl.ds(base + h * half, half)],
                sem_out,
            )
            cp.start()
            out_copies.append(cp)
        for cp in out_copies:
            cp.wait()

    return gather_kernel(t_idx, table)


def kernel(t_int, betas):
    return _sc_gather(t_int.astype(jnp.int32), betas)


# speedup vs baseline: 1.0147x; 1.0147x over previous
import functools

import jax
import jax.numpy as jnp
from jax import lax
from jax.experimental import pallas as pl
from jax.experimental.pallas import tpu as pltpu
from jax.experimental.pallas import tpu_sc as plsc

_LANES = 16


@jax.jit
def _sc_gather(t_idx, table):
    batch = t_idx.shape[0]
    table_size = table.shape[0]
    info = plsc.get_sparse_core_info()
    num_workers = info.num_subcores
    per_worker = batch // num_workers
    half = per_worker // 2

    mesh = plsc.VectorSubcoreMesh(
        core_axis_name="c", subcore_axis_name="s", num_cores=1
    )

    @functools.partial(
        pl.kernel,
        mesh=mesh,
        out_type=jax.ShapeDtypeStruct((batch,), jnp.float32),
        compiler_params=pltpu.CompilerParams(needs_layout_passes=False),
        scratch_types=[
            pltpu.VMEM((per_worker,), jnp.int32),
            pltpu.VMEM((table_size,), jnp.float32),
            pltpu.VMEM((per_worker,), jnp.float32),
            pltpu.SemaphoreType.DMA,
            pltpu.SemaphoreType.DMA,
        ],
    )
    def gather_kernel(
        t_hbm, table_hbm, out_hbm, idx_v, table_v, out_v, sem_in, sem_out
    ):
        wid = lax.axis_index("s")
        base = wid * per_worker
        cp_idx = pltpu.make_async_copy(
            t_hbm.at[pl.ds(base, per_worker)], idx_v, sem_in
        )
        cp_tab = pltpu.make_async_copy(table_hbm, table_v, sem_in)
        cp_idx.start()
        cp_tab.start()
        cp_idx.wait()
        cp_tab.wait()

        out_copies = []
        for h in range(2):
            @plsc.parallel_loop(h * half, (h + 1) * half, _LANES, unroll=16)
            def body(off):
                idx_vec = idx_v[pl.ds(off, _LANES)]
                out_v[pl.ds(off, _LANES)] = plsc.load_gather(
                    table_v, [idx_vec]
                )

            cp = pltpu.make_async_copy(
                out_v.at[pl.ds(h * half, half)],
                out_hbm.at[pl.ds(base + h * half, half)],
                sem_out,
            )
            cp.start()
            out_copies.append(cp)
        for cp in out_copies:
            cp.wait()

    return gather_kernel(t_idx, table)


def kernel(t_int, betas):
    return _sc_gather(t_int.astype(jnp.int32), betas)
